# baseline (speedup 1.0000x reference)
import functools

import jax
import jax.numpy as jnp
from jax import lax
from jax.experimental import pallas as pl
from jax.experimental.pallas import tpu as pltpu
from jax.experimental.pallas import tpu_sc as plsc

N = 16384
D = 128
LEVELS = 4
NC = 2
NS = 16
NW = NC * NS
ROWS_PER_W = N // NW
CHUNK = 64
NCHUNK = ROWS_PER_W // CHUNK
LANES = 16
NSETS = 3
NGRP = ROWS_PER_W // LANES


def _body(coords, e0, e1, e2, e3, out, idx_v, bufs_flat, sems):
    wid = lax.axis_index("s") * NC + lax.axis_index("c")
    base = wid * ROWS_PER_W
    tables = (e0, e1, e2, e3)
    bufsets = tuple(tuple(bufs_flat[s * LEVELS + l] for l in range(LEVELS))
                    for s in range(NSETS))
    gsems, wsems = sems[:NSETS], sems[NSETS:]

    for l in range(LEVELS):
        pltpu.sync_copy(coords.at[l, pl.ds(base, ROWS_PER_W)], idx_v.at[l])

    def fire_gathers(k, s):
        return [
            pltpu.async_copy(tables[l].at[idx_v.at[l, pl.ds(k * CHUNK, CHUNK)]],
                             bufsets[s][l], gsems[s])
            for l in range(LEVELS)
        ]

    gcps = [fire_gathers(0, 0), fire_gathers(1, 1), None]
    wcps = [None] * NSETS
    for k in range(NCHUNK):
        s = k % NSETS
        if k + 2 < NCHUNK:
            s2 = (k + 2) % NSETS
            with jax.named_scope("wbwait"):
                if wcps[s2] is not None:
                    wcps[s2].wait()
            with jax.named_scope("gfire"):
                gcps[s2] = fire_gathers(k + 2, s2)
        with jax.named_scope("gwait"):
            for cp in gcps[s]:
                cp.wait()
        bs = bufsets[s]

        def add_row(r, _, bs=bs):
            for col in range(D // LANES):
                sl = pl.ds(col * LANES, LANES)
                bs[0][r, sl] = bs[0][r, sl] + bs[1][r, sl] + bs[2][r, sl] + bs[3][r, sl]
            return 0

        with jax.named_scope("adds"):
            lax.fori_loop(0, CHUNK, add_row, 0)
        with jax.named_scope("wfire"):
            wcps[s] = pltpu.async_copy(
                bs[0], out.at[pl.ds(base + k * CHUNK, CHUNK)], wsems[s])
    for cp in wcps:
        if cp is not None:
            cp.wait()


def _entry(coords, e0, e1, e2, e3, out, idx_v, *rest):
    _body(coords, e0, e1, e2, e3, out, idx_v,
          rest[:NSETS * LEVELS], rest[NSETS * LEVELS:])


_mesh = plsc.VectorSubcoreMesh(core_axis_name="c", subcore_axis_name="s")

_sc_call = functools.partial(
    pl.kernel,
    mesh=_mesh,
    out_type=jax.ShapeDtypeStruct((N, D), jnp.float32),
    scratch_types=(
        [pltpu.VMEM((LEVELS, ROWS_PER_W), jnp.int32)]
        + [pltpu.VMEM((CHUNK, D), jnp.float32)] * (NSETS * LEVELS)
        + [pltpu.SemaphoreType.DMA] * (2 * NSETS)
    ),
)(_entry)


@jax.jit
def kernel(coords, emb0, emb1, emb2, emb3):
    return _sc_call(coords.T, emb0, emb1, emb2, emb3)


# speedup vs baseline: 1.0310x; 1.0310x over previous
import functools

import jax
import jax.numpy as jnp
from jax import lax
from jax.experimental import pallas as pl
from jax.experimental.pallas import tpu as pltpu
from jax.experimental.pallas import tpu_sc as plsc

N = 16384
D = 128
LEVELS = 4
NC = 2
NS = 16
NW = NC * NS
ROWS_PER_W = N // NW
CHUNK = 64
NCHUNK = ROWS_PER_W // CHUNK
LANES = 16
NSETS = 3
GROWS = LEVELS * CHUNK


def _body(idx_hbm, table, out, idx_v, b0, b1, b2, sems):
    wid = lax.axis_index("s") * NC + lax.axis_index("c")
    base = wid * ROWS_PER_W
    bufs = (b0, b1, b2)
    gsems, wsems = sems[:NSETS], sems[NSETS:]

    pltpu.sync_copy(idx_hbm.at[wid], idx_v)

    def fire_gathers(k, s):
        return [
            pltpu.async_copy(table.at[idx_v.at[2 * k + j]],
                             bufs[s].at[pl.ds(j * 2 * CHUNK, 2 * CHUNK)],
                             gsems[s])
            for j in range(2)
        ]

    gcps = [fire_gathers(0, 0), fire_gathers(1, 1), None]
    wcps = [None] * NSETS
    for k in range(NCHUNK):
        s = k % NSETS
        if k + 2 < NCHUNK:
            s2 = (k + 2) % NSETS
            with jax.named_scope("wbwait"):
                if wcps[s2] is not None:
                    wcps[s2].wait()
            gcps[s2] = fire_gathers(k + 2, s2)
        with jax.named_scope("gwait"):
            for cp in gcps[s]:
                cp.wait()
        b = bufs[s]

        def add_row(r, _, b=b):
            for col in range(D // LANES):
                sl = pl.ds(col * LANES, LANES)
                b[r, sl] = (b[r, sl] + b[r + CHUNK, sl]
                            + b[r + 2 * CHUNK, sl] + b[r + 3 * CHUNK, sl])
            return 0

        with jax.named_scope("adds"):
            lax.fori_loop(0, CHUNK, add_row, 0)
        wcps[s] = pltpu.async_copy(
            b.at[pl.ds(0, CHUNK)], out.at[pl.ds(base + k * CHUNK, CHUNK)],
            wsems[s])
    for cp in wcps:
        if cp is not None:
            cp.wait()


def _entry(idx_hbm, table, out, idx_v, b0, b1, b2, *sems):
    _body(idx_hbm, table, out, idx_v, b0, b1, b2, sems)


_mesh = plsc.VectorSubcoreMesh(core_axis_name="c", subcore_axis_name="s")

_sc_call = functools.partial(
    pl.kernel,
    mesh=_mesh,
    out_type=jax.ShapeDtypeStruct((N, D), jnp.float32),
    scratch_types=(
        [pltpu.VMEM((2 * NCHUNK, 2 * CHUNK), jnp.int32)]
        + [pltpu.VMEM((GROWS, D), jnp.float32)] * NSETS
        + [pltpu.SemaphoreType.DMA] * (2 * NSETS)
    ),
)(_entry)


@jax.jit
def kernel(coords, emb0, emb1, emb2, emb3):
    table = jnp.concatenate([emb0, emb1, emb2, emb3], axis=0)
    off = jnp.arange(LEVELS, dtype=jnp.int32) * emb0.shape[0]
    idx = coords.T.reshape(LEVELS, NW, NCHUNK, CHUNK) + off[:, None, None, None]
    idx = idx.transpose(1, 2, 0, 3).reshape(NW, 2 * NCHUNK, 2 * CHUNK)
    return _sc_call(idx, table)
